# baseline (speedup 1.0000x reference)
import functools

import jax
import jax.numpy as jnp
from jax.experimental import pallas as pl

M = 1048576
NUM_SAMPLES = 16
EPS = 1e-08
BLK = 262144
GRID = M // BLK
INV_S = 1.0 / NUM_SAMPLES


def _elbo_block(q_ref, u_top_ref, u_bot_ref, out_ref):
    i = pl.program_id(0)
    q = q_ref[0]
    p = jnp.clip(q, EPS, 1.0 - EPS)
    w = jnp.log(p) - jnp.log(1.0 - p)
    s_cnt = (jnp.sum(jnp.where(u_top_ref[...] < q, w, 0.0)) +
             jnp.sum(jnp.where(u_bot_ref[...] < q, w, 0.0)))
    s = s_cnt * INV_S - jnp.sum(p * w)

    @pl.when(i == 0)
    def _init():
        out_ref[...] = jnp.zeros((1, 1), jnp.float32)

    out_ref[...] += s


@functools.partial(jax.jit)
def _elbo(q_probs, u):
    q2 = q_probs.reshape(GRID, 1, BLK)
    out = pl.pallas_call(
        _elbo_block,
        grid=(GRID,),
        in_specs=[
            pl.BlockSpec((1, 1, BLK), lambda i: (i, 0, 0)),
            pl.BlockSpec((NUM_SAMPLES // 2, BLK), lambda i: (0, i)),
            pl.BlockSpec((NUM_SAMPLES // 2, BLK), lambda i: (1, i)),
        ],
        out_specs=pl.BlockSpec((1, 1), lambda i: (0, 0)),
        out_shape=jax.ShapeDtypeStruct((1, 1), jnp.float32),
    )(q2, u, u)
    return out[0, 0]


def kernel(q_probs, u):
    return _elbo(q_probs, u)


# speedup vs baseline: 1.0061x; 1.0061x over previous
import functools

import jax
import jax.numpy as jnp
from jax.experimental import pallas as pl

M = 1048576
NUM_SAMPLES = 16
EPS = 1e-08
BLK = 262144
GRID = M // BLK
INV_S = 1.0 / NUM_SAMPLES


def _elbo_block(q_ref, u_ref, out_ref):
    i = pl.program_id(0)
    q = q_ref[0]
    p = jnp.clip(q, EPS, 1.0 - EPS)
    w = jnp.log(p) - jnp.log(1.0 - p)
    u = u_ref[...]
    s_cnt = jnp.sum(jnp.where(u < q, w, 0.0))
    s = s_cnt * INV_S - jnp.sum(p * w)

    @pl.when(i == 0)
    def _init():
        out_ref[...] = jnp.zeros((1, 1), jnp.float32)

    out_ref[...] += s


@functools.partial(jax.jit)
def _elbo(q_probs, u):
    q2 = q_probs.reshape(GRID, 1, BLK)
    out = pl.pallas_call(
        _elbo_block,
        grid=(GRID,),
        in_specs=[
            pl.BlockSpec((1, 1, BLK), lambda i: (i, 0, 0)),
            pl.BlockSpec((NUM_SAMPLES, BLK), lambda i: (0, i)),
        ],
        out_specs=pl.BlockSpec((1, 1), lambda i: (0, 0)),
        out_shape=jax.ShapeDtypeStruct((1, 1), jnp.float32),
    )(q2, u)
    return out[0, 0]


def kernel(q_probs, u):
    return _elbo(q_probs, u)
